# baseline (speedup 1.0000x reference)
import functools

import jax
import jax.numpy as jnp
from jax import lax
from jax.experimental import pallas as pl
from jax.experimental.pallas import tpu as pltpu
from jax.experimental.pallas import tpu_sc as plsc

N = 100000
H = 128
NF = 2000
MIN_FRAG = 3.0
BLK = 2560
GRID_A = 40
NPAD = 102400
NFP = 2048
NWIN = NFP // H
QW = 16
NWORK = 32
CHUNK = 128
CHUNKS_PER_W = NPAD // (NWORK * CHUNK)
DENOM = float(NF) * NF - NF + 1e-6


def _stage_a_body(ids_ref, s_ref, v_ref, g_ref, b_ref, w1_ref, b1_ref, w2_ref,
                  os_ref, on_ref, oq_ref):
    p = pl.program_id(0)

    @pl.when(p == 0)
    def _init():
        os_ref[...] = jnp.zeros_like(os_ref)
        on_ref[...] = jnp.zeros_like(on_ref)

    x = s_ref[...]
    mu = jnp.mean(x, axis=-1, keepdims=True)
    xc = x - mu
    var = jnp.mean(xc * xc, axis=-1, keepdims=True)
    xn = xc / jnp.sqrt(var + 1e-5) * g_ref[...] + b_ref[...]
    pre = lax.dot_general(xn, w1_ref[...], (((1,), (1,)), ((), ())),
                          preferred_element_type=jnp.float32) + b1_ref[...]
    s = pre * jax.nn.sigmoid(pre)
    n2 = jnp.sum(s * s, axis=-1, keepdims=True)
    ns = s / jnp.sqrt(jnp.maximum(n2, 1e-24))

    v0 = v_ref[:, 0, :]
    v1 = v_ref[:, 1, :]
    v2 = v_ref[:, 2, :]
    r0 = jnp.sum(v0 * v0, axis=-1, keepdims=True)
    r1 = jnp.sum(v1 * v1, axis=-1, keepdims=True)
    r2 = jnp.sum(v2 * v2, axis=-1, keepdims=True)
    vmag = (jnp.sqrt(r0) + jnp.sqrt(r1) + jnp.sqrt(r2)) * (1.0 / 3.0)

    w2r0 = jnp.sum(w2_ref[0:1, :])
    pv = vmag * w2r0
    v = pv * jax.nn.sigmoid(pv)
    nv = v / jnp.maximum(jnp.sqrt(jnp.maximum(v * v, 1e-24)), 1e-12)

    row = p * BLK + lax.broadcasted_iota(jnp.int32, (BLK, 1), 0)
    mask = row < N
    s = jnp.where(mask, s, 0.0)
    ns = jnp.where(mask, ns, 0.0)

    lane16 = lax.broadcasted_iota(jnp.int32, (BLK, H), 1)
    quart = (jnp.where(lane16 == 0, vmag, 0.0)
             + jnp.where(lane16 == 1, v, 0.0)
             + jnp.where(lane16 == 2, nv, 0.0)
             + jnp.where(lane16 == 3, 1.0, 0.0))
    oq_ref[...] = jnp.where(mask, quart, 0.0).reshape(BLK * H)

    idsl = ids_ref[0]
    rows = BLK // H
    rr = lax.broadcasted_iota(jnp.int32, (BLK, rows), 0) // H
    rc = lax.broadcasted_iota(jnp.int32, (BLK, rows), 1)
    exp = (rr == rc).astype(jnp.float32)
    mrep = lax.dot_general(exp, idsl, (((1,), (0,)), ((), ())),
                           precision=lax.Precision.HIGHEST,
                           preferred_element_type=jnp.float32)
    lane = lax.broadcasted_iota(jnp.int32, (BLK, H), 1)
    imod = lax.broadcasted_iota(jnp.int32, (BLK, 1), 0) & (H - 1)
    ids = jnp.sum(jnp.where(lane == imod, mrep, 0.0), axis=-1, keepdims=True)
    idmin = jnp.min(ids)
    idmax = jnp.max(ids)
    for k in range(NWIN):
        @pl.when((idmin <= float(k * H + (H - 1))) & (idmax >= float(k * H)))
        def _win(k=k):
            oh = (ids == (k * H + lane).astype(jnp.float32)).astype(jnp.float32)
            cs = lax.dot_general(oh, s, (((0,), (0,)), ((), ())),
                                 preferred_element_type=jnp.float32)
            cn = lax.dot_general(oh, ns, (((0,), (0,)), ((), ())),
                                 preferred_element_type=jnp.float32)
            os_ref[k * H:(k + 1) * H, :] += cs
            on_ref[k * H:(k + 1) * H, :] += cn


_stage_a_call = pl.pallas_call(
    _stage_a_body,
    grid=(GRID_A,),
    in_specs=[
        pl.BlockSpec((1, BLK // H, H), lambda i: (i, 0, 0)),
        pl.BlockSpec((BLK, H), lambda i: (i, 0)),
        pl.BlockSpec((BLK, 3, H), lambda i: (i, 0, 0)),
        pl.BlockSpec((1, H), lambda i: (0, 0)),
        pl.BlockSpec((1, H), lambda i: (0, 0)),
        pl.BlockSpec((H, H), lambda i: (0, 0)),
        pl.BlockSpec((1, H), lambda i: (0, 0)),
        pl.BlockSpec((H, H), lambda i: (0, 0)),
    ],
    out_specs=[
        pl.BlockSpec((NFP, H), lambda i: (0, 0)),
        pl.BlockSpec((NFP, H), lambda i: (0, 0)),
        pl.BlockSpec((BLK * H,), lambda i: (i,)),
    ],
    out_shape=[
        jax.ShapeDtypeStruct((NFP, H), jnp.float32),
        jax.ShapeDtypeStruct((NFP, H), jnp.float32),
        jax.ShapeDtypeStruct((NPAD * H,), jnp.float32),
    ],
)


@functools.lru_cache(maxsize=1)
def _make_stage_b():
    mesh = plsc.VectorSubcoreMesh(core_axis_name="c", subcore_axis_name="s",
                                  num_cores=2, num_subcores=16)

    def body(quart, ids2d, zeros, out, rowbuf, idvbuf, acc):
        cid = lax.axis_index("c")
        sid = lax.axis_index("s")
        wid = cid * 16 + sid
        pltpu.sync_copy(zeros, acc)

        def chunk_step(i, carry):
            chunk = wid * CHUNKS_PER_W + i
            pltpu.sync_copy(
                quart.at[pl.ds(pl.multiple_of(chunk * CHUNK * H, CHUNK * H), CHUNK * H)],
                rowbuf)
            pltpu.sync_copy(ids2d.at[chunk], idvbuf)

            def grp_step(g, carry2):
                idvec = idvbuf[pl.ds(g * 16, 16)]
                for j in range(16):
                    fid = idvec[j]
                    off = pl.multiple_of(fid * QW, QW)
                    acc[pl.ds(off, QW)] = (acc[pl.ds(off, QW)]
                                           + rowbuf[pl.ds((g * 16 + j) * H, QW)])
                return carry2

            lax.fori_loop(0, CHUNK // 16, grp_step, 0)
            return carry

        lax.fori_loop(0, CHUNKS_PER_W, chunk_step, 0)
        pltpu.sync_copy(acc, out.at[pl.ds(pl.multiple_of(wid * NFP * QW, NFP * QW), NFP * QW)])

    return pl.kernel(
        body,
        out_type=jax.ShapeDtypeStruct((NWORK * NFP * QW,), jnp.float32),
        mesh=mesh,
        scratch_types=[
            pltpu.VMEM((CHUNK * H,), jnp.float32),
            pltpu.VMEM((CHUNK,), jnp.int32),
            pltpu.VMEM((NFP * QW,), jnp.float32),
        ],
    )


def _finalize_body(s_ref, n_ref, q_ref, o_ref):
    rpw = NFP * QW // H
    qs = q_ref[0:rpw, :]
    for w in range(1, NWORK):
        qs = qs + q_ref[w * rpw:(w + 1) * rpw, :]
    frow = lax.broadcasted_iota(jnp.int32, (NFP, rpw), 0) // 8
    rcol = lax.broadcasted_iota(jnp.int32, (NFP, rpw), 1)
    expand = (frow == rcol).astype(jnp.float32)
    m = lax.dot_general(expand, qs, (((1,), (0,)), ((), ())),
                        precision=lax.Precision.HIGHEST,
                        preferred_element_type=jnp.float32)
    fmod8 = lax.broadcasted_iota(jnp.int32, (NFP, 1), 0) & 7
    lane = lax.broadcasted_iota(jnp.int32, (NFP, H), 1)
    ssum = s_ref[...]
    nssum = n_ref[...]

    def pick(k):
        return jnp.sum(jnp.where(lane == fmod8 * QW + k, m, 0.0),
                       axis=-1, keepdims=True)

    vmagsum = pick(0)
    vsum = pick(1)
    nvsum = pick(2)
    c = pick(3)

    valid = (c >= MIN_FRAG).astype(jnp.float32)
    cm = jnp.maximum(c, 1.0)

    fmean = valid * ssum / cm
    mn = fmean / jnp.sqrt(jnp.maximum(jnp.sum(fmean * fmean, -1, keepdims=True), 1e-24))
    simsum = jnp.sum(nssum * mn, -1, keepdims=True)
    dev_mean = valid * (c - simsum) / cm
    w = c * valid
    wsum = jnp.sum(w)
    loss1 = jnp.sum(dev_mean * w * valid) / wsum

    vfmean = valid * vsum / cm
    vmn = vfmean / jnp.maximum(jnp.sqrt(jnp.maximum(vfmean * vfmean, 1e-24)), 1e-12)
    vdev_mean = valid * (c - vmn * nvsum) / cm
    loss2 = jnp.sum(vdev_mean * w * valid) / wsum

    nz = (c > 0).astype(jnp.float32)
    sfrag = nz * ssum / cm
    vfrag = nz * vmagsum / cm
    n2 = jnp.sum(sfrag * sfrag, -1, keepdims=True) + vfrag * vfrag
    inv = 1.0 / jnp.sqrt(jnp.maximum(n2, 1e-24))
    sn = sfrag * inv
    vn = vfrag * inv
    s128 = jnp.sum(sn, axis=0, keepdims=True)
    sv = jnp.sum(vn)
    ss = jnp.sum(s128 * s128) + sv * sv
    t = jnp.sum(sn * sn) + jnp.sum(vn * vn)
    inter = (ss - t) / DENOM

    total = 0.03 * (loss1 + loss2 + 0.2 * inter) * 0.05
    o_ref[...] = jnp.reshape(total, (1, 1))


_finalize_call = pl.pallas_call(
    _finalize_body,
    in_specs=[
        pl.BlockSpec((NFP, H), lambda: (0, 0)),
        pl.BlockSpec((NFP, H), lambda: (0, 0)),
        pl.BlockSpec((NWORK * NFP * QW // H, H), lambda: (0, 0)),
    ],
    out_specs=pl.BlockSpec((1, 1), lambda: (0, 0)),
    out_shape=jax.ShapeDtypeStruct((1, 1), jnp.float32),
)


def kernel(scalar_short, scalar_long, vector_short, vector_long, fragment_ids,
           ln_gamma, ln_beta, W1, b1, W2):
    ids_pad = jnp.pad(fragment_ids, (0, NPAD - N))
    ids_f = ids_pad.astype(jnp.float32).reshape(GRID_A, BLK // H, H)
    ssum, nssum, quart = _stage_a_call(
        ids_f, scalar_short, vector_short,
        ln_gamma.reshape(1, H), ln_beta.reshape(1, H),
        W1, b1.reshape(1, H), W2,
    )
    ids2d = ids_pad.reshape(NPAD // CHUNK, CHUNK)
    zeros = jnp.zeros((NFP * QW,), jnp.float32)
    qacc = _make_stage_b()(quart, ids2d, zeros)
    out = _finalize_call(ssum, nssum, qacc.reshape(NWORK * NFP * QW // H, H))
    return jnp.reshape(out, ())


# speedup vs baseline: 2.0416x; 2.0416x over previous
import functools

import jax
import jax.numpy as jnp
from jax import lax
from jax.experimental import pallas as pl
from jax.experimental.pallas import tpu as pltpu
from jax.experimental.pallas import tpu_sc as plsc

N = 100000
H = 128
NF = 2000
MIN_FRAG = 3.0
BLK = 2560
GRID_A = 40
NPAD = 102400
NFP = 2048
NWIN = NFP // H
QW = 16
NWORK = 32
CHUNK = 128
CHUNKS_PER_W = NPAD // (NWORK * CHUNK)
DENOM = float(NF) * NF - NF + 1e-6


def _stage_a_body(ids_ref, s_ref, v_ref, g_ref, b_ref, w1_ref, b1_ref, w2_ref,
                  os_ref, on_ref, oq_ref):
    p = pl.program_id(0)

    @pl.when(p == 0)
    def _init():
        os_ref[...] = jnp.zeros_like(os_ref)
        on_ref[...] = jnp.zeros_like(on_ref)

    x = s_ref[...]
    mu = jnp.mean(x, axis=-1, keepdims=True)
    xc = x - mu
    var = jnp.mean(xc * xc, axis=-1, keepdims=True)
    xn = xc / jnp.sqrt(var + 1e-5) * g_ref[...] + b_ref[...]
    pre = lax.dot_general(xn, w1_ref[...], (((1,), (1,)), ((), ())),
                          preferred_element_type=jnp.float32) + b1_ref[...]
    s = pre * jax.nn.sigmoid(pre)
    n2 = jnp.sum(s * s, axis=-1, keepdims=True)
    ns = s / jnp.sqrt(jnp.maximum(n2, 1e-24))

    v0 = v_ref[0]
    v1 = v_ref[1]
    v2 = v_ref[2]
    r0 = jnp.sum(v0 * v0, axis=-1, keepdims=True)
    r1 = jnp.sum(v1 * v1, axis=-1, keepdims=True)
    r2 = jnp.sum(v2 * v2, axis=-1, keepdims=True)
    vmag = (jnp.sqrt(r0) + jnp.sqrt(r1) + jnp.sqrt(r2)) * (1.0 / 3.0)

    w2r0 = jnp.sum(w2_ref[0:1, :])
    pv = vmag * w2r0
    v = pv * jax.nn.sigmoid(pv)
    nv = v / jnp.maximum(jnp.sqrt(jnp.maximum(v * v, 1e-24)), 1e-12)

    row = p * BLK + lax.broadcasted_iota(jnp.int32, (BLK, 1), 0)
    mask = row < N
    s = jnp.where(mask, s, 0.0)
    ns = jnp.where(mask, ns, 0.0)

    lane16 = lax.broadcasted_iota(jnp.int32, (BLK, H), 1)
    quart = (jnp.where(lane16 == 0, vmag, 0.0)
             + jnp.where(lane16 == 1, v, 0.0)
             + jnp.where(lane16 == 2, nv, 0.0)
             + jnp.where(lane16 == 3, 1.0, 0.0))
    oq_ref[...] = jnp.where(mask, quart, 0.0).reshape(BLK * H)

    idsl = ids_ref[0]
    rows = BLK // H
    rr = lax.broadcasted_iota(jnp.int32, (BLK, rows), 0) // H
    rc = lax.broadcasted_iota(jnp.int32, (BLK, rows), 1)
    exp = (rr == rc).astype(jnp.float32)
    mrep = lax.dot_general(exp, idsl, (((1,), (0,)), ((), ())),
                           precision=lax.Precision.HIGHEST,
                           preferred_element_type=jnp.float32)
    lane = lax.broadcasted_iota(jnp.int32, (BLK, H), 1)
    imod = lax.broadcasted_iota(jnp.int32, (BLK, 1), 0) & (H - 1)
    ids = jnp.sum(jnp.where(lane == imod, mrep, 0.0), axis=-1, keepdims=True)
    idmin = jnp.min(ids)
    idmax = jnp.max(ids)
    for k in range(NWIN):
        @pl.when((idmin <= float(k * H + (H - 1))) & (idmax >= float(k * H)))
        def _win(k=k):
            oh = (ids == (k * H + lane).astype(jnp.float32)).astype(jnp.float32)
            cs = lax.dot_general(oh, s, (((0,), (0,)), ((), ())),
                                 preferred_element_type=jnp.float32)
            cn = lax.dot_general(oh, ns, (((0,), (0,)), ((), ())),
                                 preferred_element_type=jnp.float32)
            os_ref[k * H:(k + 1) * H, :] += cs
            on_ref[k * H:(k + 1) * H, :] += cn


_stage_a_call = pl.pallas_call(
    _stage_a_body,
    grid=(GRID_A,),
    in_specs=[
        pl.BlockSpec((1, BLK // H, H), lambda i: (i, 0, 0)),
        pl.BlockSpec((BLK, H), lambda i: (i, 0)),
        pl.BlockSpec((3, BLK, H), lambda i: (0, i, 0)),
        pl.BlockSpec((1, H), lambda i: (0, 0)),
        pl.BlockSpec((1, H), lambda i: (0, 0)),
        pl.BlockSpec((H, H), lambda i: (0, 0)),
        pl.BlockSpec((1, H), lambda i: (0, 0)),
        pl.BlockSpec((H, H), lambda i: (0, 0)),
    ],
    out_specs=[
        pl.BlockSpec((NFP, H), lambda i: (0, 0)),
        pl.BlockSpec((NFP, H), lambda i: (0, 0)),
        pl.BlockSpec((BLK * H,), lambda i: (i,)),
    ],
    out_shape=[
        jax.ShapeDtypeStruct((NFP, H), jnp.float32),
        jax.ShapeDtypeStruct((NFP, H), jnp.float32),
        jax.ShapeDtypeStruct((NPAD * H,), jnp.float32),
    ],
)


@functools.lru_cache(maxsize=1)
def _make_stage_b():
    mesh = plsc.VectorSubcoreMesh(core_axis_name="c", subcore_axis_name="s",
                                  num_cores=2, num_subcores=16)

    def body(quart, ids2d, zeros, out, rowbuf, idvbuf, acc):
        cid = lax.axis_index("c")
        sid = lax.axis_index("s")
        wid = cid * 16 + sid
        pltpu.sync_copy(zeros, acc)

        def chunk_step(i, carry):
            chunk = wid * CHUNKS_PER_W + i
            pltpu.sync_copy(
                quart.at[pl.ds(pl.multiple_of(chunk * CHUNK * H, CHUNK * H), CHUNK * H)],
                rowbuf)
            pltpu.sync_copy(ids2d.at[chunk], idvbuf)

            def grp_step(g, carry2):
                idvec = idvbuf[pl.ds(g * 16, 16)]
                for j in range(16):
                    fid = idvec[j]
                    off = pl.multiple_of(fid * QW, QW)
                    acc[pl.ds(off, QW)] = (acc[pl.ds(off, QW)]
                                           + rowbuf[pl.ds((g * 16 + j) * H, QW)])
                return carry2

            lax.fori_loop(0, CHUNK // 16, grp_step, 0)
            return carry

        lax.fori_loop(0, CHUNKS_PER_W, chunk_step, 0)
        pltpu.sync_copy(acc, out.at[pl.ds(pl.multiple_of(wid * NFP * QW, NFP * QW), NFP * QW)])

    return pl.kernel(
        body,
        out_type=jax.ShapeDtypeStruct((NWORK * NFP * QW,), jnp.float32),
        mesh=mesh,
        scratch_types=[
            pltpu.VMEM((CHUNK * H,), jnp.float32),
            pltpu.VMEM((CHUNK,), jnp.int32),
            pltpu.VMEM((NFP * QW,), jnp.float32),
        ],
    )


def _finalize_body(s_ref, n_ref, q_ref, o_ref):
    rpw = NFP * QW // H
    qs = q_ref[0:rpw, :]
    for w in range(1, NWORK):
        qs = qs + q_ref[w * rpw:(w + 1) * rpw, :]
    frow = lax.broadcasted_iota(jnp.int32, (NFP, rpw), 0) // 8
    rcol = lax.broadcasted_iota(jnp.int32, (NFP, rpw), 1)
    expand = (frow == rcol).astype(jnp.float32)
    m = lax.dot_general(expand, qs, (((1,), (0,)), ((), ())),
                        precision=lax.Precision.HIGHEST,
                        preferred_element_type=jnp.float32)
    fmod8 = lax.broadcasted_iota(jnp.int32, (NFP, 1), 0) & 7
    lane = lax.broadcasted_iota(jnp.int32, (NFP, H), 1)
    ssum = s_ref[...]
    nssum = n_ref[...]

    def pick(k):
        return jnp.sum(jnp.where(lane == fmod8 * QW + k, m, 0.0),
                       axis=-1, keepdims=True)

    vmagsum = pick(0)
    vsum = pick(1)
    nvsum = pick(2)
    c = pick(3)

    valid = (c >= MIN_FRAG).astype(jnp.float32)
    cm = jnp.maximum(c, 1.0)

    fmean = valid * ssum / cm
    mn = fmean / jnp.sqrt(jnp.maximum(jnp.sum(fmean * fmean, -1, keepdims=True), 1e-24))
    simsum = jnp.sum(nssum * mn, -1, keepdims=True)
    dev_mean = valid * (c - simsum) / cm
    w = c * valid
    wsum = jnp.sum(w)
    loss1 = jnp.sum(dev_mean * w * valid) / wsum

    vfmean = valid * vsum / cm
    vmn = vfmean / jnp.maximum(jnp.sqrt(jnp.maximum(vfmean * vfmean, 1e-24)), 1e-12)
    vdev_mean = valid * (c - vmn * nvsum) / cm
    loss2 = jnp.sum(vdev_mean * w * valid) / wsum

    nz = (c > 0).astype(jnp.float32)
    sfrag = nz * ssum / cm
    vfrag = nz * vmagsum / cm
    n2 = jnp.sum(sfrag * sfrag, -1, keepdims=True) + vfrag * vfrag
    inv = 1.0 / jnp.sqrt(jnp.maximum(n2, 1e-24))
    sn = sfrag * inv
    vn = vfrag * inv
    s128 = jnp.sum(sn, axis=0, keepdims=True)
    sv = jnp.sum(vn)
    ss = jnp.sum(s128 * s128) + sv * sv
    t = jnp.sum(sn * sn) + jnp.sum(vn * vn)
    inter = (ss - t) / DENOM

    total = 0.03 * (loss1 + loss2 + 0.2 * inter) * 0.05
    o_ref[...] = jnp.reshape(total, (1, 1))


_finalize_call = pl.pallas_call(
    _finalize_body,
    in_specs=[
        pl.BlockSpec((NFP, H), lambda: (0, 0)),
        pl.BlockSpec((NFP, H), lambda: (0, 0)),
        pl.BlockSpec((NWORK * NFP * QW // H, H), lambda: (0, 0)),
    ],
    out_specs=pl.BlockSpec((1, 1), lambda: (0, 0)),
    out_shape=jax.ShapeDtypeStruct((1, 1), jnp.float32),
)


def kernel(scalar_short, scalar_long, vector_short, vector_long, fragment_ids,
           ln_gamma, ln_beta, W1, b1, W2):
    ids_pad = jnp.pad(fragment_ids, (0, NPAD - N))
    ids_f = ids_pad.astype(jnp.float32).reshape(GRID_A, BLK // H, H)
    ssum, nssum, quart = _stage_a_call(
        ids_f, scalar_short, vector_short.transpose(1, 0, 2),
        ln_gamma.reshape(1, H), ln_beta.reshape(1, H),
        W1, b1.reshape(1, H), W2,
    )
    ids2d = ids_pad.reshape(NPAD // CHUNK, CHUNK)
    zeros = jnp.zeros((NFP * QW,), jnp.float32)
    qacc = _make_stage_b()(quart, ids2d, zeros)
    out = _finalize_call(ssum, nssum, qacc.reshape(NWORK * NFP * QW // H, H))
    return jnp.reshape(out, ())


# speedup vs baseline: 2.1309x; 1.0437x over previous
import functools

import jax
import jax.numpy as jnp
from jax import lax
from jax.experimental import pallas as pl
from jax.experimental.pallas import tpu as pltpu
from jax.experimental.pallas import tpu_sc as plsc

N = 100000
H = 128
NF = 2000
MIN_FRAG = 3.0
BLK = 2560
GRID_A = 40
NPAD = 102400
NFP = 2048
NWIN = NFP // H
QW = 16
NWORK = 32
CHUNK = 320
CHUNKS_PER_W = NPAD // (NWORK * CHUNK)
DENOM = float(NF) * NF - NF + 1e-6


def _stage_a_body(ids_ref, s_ref, v_ref, g_ref, b_ref, w1_ref, b1_ref, w2_ref,
                  os_ref, on_ref, oq_ref):
    p = pl.program_id(0)

    @pl.when(p == 0)
    def _init():
        os_ref[...] = jnp.zeros_like(os_ref)
        on_ref[...] = jnp.zeros_like(on_ref)

    x = s_ref[...]
    mu = jnp.mean(x, axis=-1, keepdims=True)
    xc = x - mu
    var = jnp.mean(xc * xc, axis=-1, keepdims=True)
    xn = xc / jnp.sqrt(var + 1e-5) * g_ref[...] + b_ref[...]
    pre = lax.dot_general(xn, w1_ref[...], (((1,), (1,)), ((), ())),
                          preferred_element_type=jnp.float32) + b1_ref[...]
    s = pre * jax.nn.sigmoid(pre)
    n2 = jnp.sum(s * s, axis=-1, keepdims=True)
    ns = s / jnp.sqrt(jnp.maximum(n2, 1e-24))

    v0 = v_ref[0]
    v1 = v_ref[1]
    v2 = v_ref[2]
    r0 = jnp.sum(v0 * v0, axis=-1, keepdims=True)
    r1 = jnp.sum(v1 * v1, axis=-1, keepdims=True)
    r2 = jnp.sum(v2 * v2, axis=-1, keepdims=True)
    vmag = (jnp.sqrt(r0) + jnp.sqrt(r1) + jnp.sqrt(r2)) * (1.0 / 3.0)

    w2r0 = jnp.sum(w2_ref[0:1, :])
    pv = vmag * w2r0
    v = pv * jax.nn.sigmoid(pv)
    nv = v / jnp.maximum(jnp.sqrt(jnp.maximum(v * v, 1e-24)), 1e-12)

    row = p * BLK + lax.broadcasted_iota(jnp.int32, (BLK, 1), 0)
    mask = row < N
    s = jnp.where(mask, s, 0.0)
    ns = jnp.where(mask, ns, 0.0)

    lane16 = lax.broadcasted_iota(jnp.int32, (BLK, H), 1)
    quart = (jnp.where(lane16 == 0, vmag, 0.0)
             + jnp.where(lane16 == 1, v, 0.0)
             + jnp.where(lane16 == 2, nv, 0.0)
             + jnp.where(lane16 == 3, 1.0, 0.0))
    oq_ref[...] = jnp.where(mask, quart, 0.0).reshape(BLK * H)

    idsl = ids_ref[0]
    rows = BLK // H
    rr = lax.broadcasted_iota(jnp.int32, (BLK, rows), 0) // H
    rc = lax.broadcasted_iota(jnp.int32, (BLK, rows), 1)
    exp = (rr == rc).astype(jnp.float32)
    mrep = lax.dot_general(exp, idsl, (((1,), (0,)), ((), ())),
                           precision=lax.Precision.HIGHEST,
                           preferred_element_type=jnp.float32)
    lane = lax.broadcasted_iota(jnp.int32, (BLK, H), 1)
    imod = lax.broadcasted_iota(jnp.int32, (BLK, 1), 0) & (H - 1)
    ids = jnp.sum(jnp.where(lane == imod, mrep, 0.0), axis=-1, keepdims=True)
    idmin = jnp.min(ids)
    idmax = jnp.max(ids)
    for k in range(NWIN):
        @pl.when((idmin <= float(k * H + (H - 1))) & (idmax >= float(k * H)))
        def _win(k=k):
            oh = (ids == (k * H + lane).astype(jnp.float32)).astype(jnp.float32)
            cs = lax.dot_general(oh, s, (((0,), (0,)), ((), ())),
                                 preferred_element_type=jnp.float32)
            cn = lax.dot_general(oh, ns, (((0,), (0,)), ((), ())),
                                 preferred_element_type=jnp.float32)
            os_ref[k * H:(k + 1) * H, :] += cs
            on_ref[k * H:(k + 1) * H, :] += cn


_stage_a_call = pl.pallas_call(
    _stage_a_body,
    grid=(GRID_A,),
    in_specs=[
        pl.BlockSpec((1, BLK // H, H), lambda i: (i, 0, 0)),
        pl.BlockSpec((BLK, H), lambda i: (i, 0)),
        pl.BlockSpec((3, BLK, H), lambda i: (0, i, 0)),
        pl.BlockSpec((1, H), lambda i: (0, 0)),
        pl.BlockSpec((1, H), lambda i: (0, 0)),
        pl.BlockSpec((H, H), lambda i: (0, 0)),
        pl.BlockSpec((1, H), lambda i: (0, 0)),
        pl.BlockSpec((H, H), lambda i: (0, 0)),
    ],
    out_specs=[
        pl.BlockSpec((NFP, H), lambda i: (0, 0)),
        pl.BlockSpec((NFP, H), lambda i: (0, 0)),
        pl.BlockSpec((BLK * H,), lambda i: (i,)),
    ],
    out_shape=[
        jax.ShapeDtypeStruct((NFP, H), jnp.float32),
        jax.ShapeDtypeStruct((NFP, H), jnp.float32),
        jax.ShapeDtypeStruct((NPAD * H,), jnp.float32),
    ],
)


@functools.lru_cache(maxsize=1)
def _make_stage_b():
    mesh = plsc.VectorSubcoreMesh(core_axis_name="c", subcore_axis_name="s",
                                  num_cores=2, num_subcores=16)

    def body(quart, ids1d, zeros, out, rowbuf, idvbuf, acc):
        cid = lax.axis_index("c")
        sid = lax.axis_index("s")
        wid = cid * 16 + sid
        pltpu.sync_copy(zeros, acc)

        def chunk_step(i, carry):
            chunk = wid * CHUNKS_PER_W + i
            pltpu.sync_copy(
                quart.at[pl.ds(pl.multiple_of(chunk * CHUNK * H, CHUNK * H), CHUNK * H)],
                rowbuf)
            pltpu.sync_copy(
                ids1d.at[pl.ds(pl.multiple_of(chunk * CHUNK, CHUNK), CHUNK)], idvbuf)

            def grp_step(g, carry2):
                idvec = idvbuf[pl.ds(g * 16, 16)]
                for j in range(16):
                    fid = idvec[j]
                    off = pl.multiple_of(fid * QW, QW)
                    acc[pl.ds(off, QW)] = (acc[pl.ds(off, QW)]
                                           + rowbuf[pl.ds((g * 16 + j) * H, QW)])
                return carry2

            lax.fori_loop(0, CHUNK // 16, grp_step, 0)
            return carry

        lax.fori_loop(0, CHUNKS_PER_W, chunk_step, 0)
        pltpu.sync_copy(acc, out.at[pl.ds(pl.multiple_of(wid * NFP * QW, NFP * QW), NFP * QW)])

    return pl.kernel(
        body,
        out_type=jax.ShapeDtypeStruct((NWORK * NFP * QW,), jnp.float32),
        mesh=mesh,
        scratch_types=[
            pltpu.VMEM((CHUNK * H,), jnp.float32),
            pltpu.VMEM((CHUNK,), jnp.int32),
            pltpu.VMEM((NFP * QW,), jnp.float32),
        ],
    )


def _finalize_body(s_ref, n_ref, q_ref, o_ref):
    rpw = NFP * QW // H
    qs = q_ref[0:rpw, :]
    for w in range(1, NWORK):
        qs = qs + q_ref[w * rpw:(w + 1) * rpw, :]
    frow = lax.broadcasted_iota(jnp.int32, (NFP, rpw), 0) // 8
    rcol = lax.broadcasted_iota(jnp.int32, (NFP, rpw), 1)
    expand = (frow == rcol).astype(jnp.float32)
    m = lax.dot_general(expand, qs, (((1,), (0,)), ((), ())),
                        precision=lax.Precision.HIGHEST,
                        preferred_element_type=jnp.float32)
    fmod8 = lax.broadcasted_iota(jnp.int32, (NFP, 1), 0) & 7
    lane = lax.broadcasted_iota(jnp.int32, (NFP, H), 1)
    ssum = s_ref[...]
    nssum = n_ref[...]

    def pick(k):
        return jnp.sum(jnp.where(lane == fmod8 * QW + k, m, 0.0),
                       axis=-1, keepdims=True)

    vmagsum = pick(0)
    vsum = pick(1)
    nvsum = pick(2)
    c = pick(3)

    valid = (c >= MIN_FRAG).astype(jnp.float32)
    cm = jnp.maximum(c, 1.0)

    fmean = valid * ssum / cm
    mn = fmean / jnp.sqrt(jnp.maximum(jnp.sum(fmean * fmean, -1, keepdims=True), 1e-24))
    simsum = jnp.sum(nssum * mn, -1, keepdims=True)
    dev_mean = valid * (c - simsum) / cm
    w = c * valid
    wsum = jnp.sum(w)
    loss1 = jnp.sum(dev_mean * w * valid) / wsum

    vfmean = valid * vsum / cm
    vmn = vfmean / jnp.maximum(jnp.sqrt(jnp.maximum(vfmean * vfmean, 1e-24)), 1e-12)
    vdev_mean = valid * (c - vmn * nvsum) / cm
    loss2 = jnp.sum(vdev_mean * w * valid) / wsum

    nz = (c > 0).astype(jnp.float32)
    sfrag = nz * ssum / cm
    vfrag = nz * vmagsum / cm
    n2 = jnp.sum(sfrag * sfrag, -1, keepdims=True) + vfrag * vfrag
    inv = 1.0 / jnp.sqrt(jnp.maximum(n2, 1e-24))
    sn = sfrag * inv
    vn = vfrag * inv
    s128 = jnp.sum(sn, axis=0, keepdims=True)
    sv = jnp.sum(vn)
    ss = jnp.sum(s128 * s128) + sv * sv
    t = jnp.sum(sn * sn) + jnp.sum(vn * vn)
    inter = (ss - t) / DENOM

    total = 0.03 * (loss1 + loss2 + 0.2 * inter) * 0.05
    o_ref[...] = jnp.reshape(total, (1, 1))


_finalize_call = pl.pallas_call(
    _finalize_body,
    in_specs=[
        pl.BlockSpec((NFP, H), lambda: (0, 0)),
        pl.BlockSpec((NFP, H), lambda: (0, 0)),
        pl.BlockSpec((NWORK * NFP * QW // H, H), lambda: (0, 0)),
    ],
    out_specs=pl.BlockSpec((1, 1), lambda: (0, 0)),
    out_shape=jax.ShapeDtypeStruct((1, 1), jnp.float32),
)


def kernel(scalar_short, scalar_long, vector_short, vector_long, fragment_ids,
           ln_gamma, ln_beta, W1, b1, W2):
    ids_pad = jnp.pad(fragment_ids, (0, NPAD - N))
    ids_f = ids_pad.astype(jnp.float32).reshape(GRID_A, BLK // H, H)
    ssum, nssum, quart = _stage_a_call(
        ids_f, scalar_short, vector_short.transpose(1, 0, 2),
        ln_gamma.reshape(1, H), ln_beta.reshape(1, H),
        W1, b1.reshape(1, H), W2,
    )
    zeros = jnp.zeros((NFP * QW,), jnp.float32)
    qacc = _make_stage_b()(quart, ids_pad, zeros)
    out = _finalize_call(ssum, nssum, qacc.reshape(NWORK * NFP * QW // H, H))
    return jnp.reshape(out, ())
